# Initial kernel scaffold; baseline (speedup 1.0000x reference)
import jax
import jax.numpy as jnp
from jax.experimental import pallas as pl


def kernel(x, codebook):
    raise NotImplementedError("write your pallas kernel here")


# speedup vs baseline: 5.5320x; 5.5320x over previous
import jax
import jax.numpy as jnp
from jax.experimental import pallas as pl
from jax.experimental.pallas import tpu as pltpu


def _vq_band_kernel(x_ref, cb_ref, out_ref):
    xb = x_ref[0, 0]
    cb = cb_ref[0]
    dots = jax.lax.dot_general(
        cb, xb, (((1,), (0,)), ((), ())),
        preferred_element_type=jnp.float32)
    cb_sq = jnp.sum(cb * cb, axis=1, keepdims=True)
    score = cb_sq - 2.0 * dots
    idx = jnp.argmin(score, axis=0)
    row_ids = jax.lax.broadcasted_iota(jnp.int32, score.shape, 0)
    onehot = (row_ids == idx[None, :]).astype(jnp.float32)
    out = jax.lax.dot_general(
        cb, onehot, (((0,), (0,)), ((), ())),
        preferred_element_type=jnp.float32,
        precision=jax.lax.Precision.HIGHEST)
    out_ref[0, 0] = out


def kernel(x, codebook):
    batch, n_band, n_chan, time = x.shape
    num_code = codebook.shape[1]
    grid = (n_band, batch)
    return pl.pallas_call(
        _vq_band_kernel,
        grid=grid,
        in_specs=[
            pl.BlockSpec((1, 1, n_chan, time), lambda n, b: (b, n, 0, 0)),
            pl.BlockSpec((1, num_code, n_chan), lambda n, b: (n, 0, 0)),
        ],
        out_specs=pl.BlockSpec((1, 1, n_chan, time), lambda n, b: (b, n, 0, 0)),
        out_shape=jax.ShapeDtypeStruct(x.shape, x.dtype),
        compiler_params=pltpu.CompilerParams(
            dimension_semantics=("arbitrary", "arbitrary"),
        ),
    )(x, codebook)


# speedup vs baseline: 11.5541x; 2.0886x over previous
import jax
import jax.numpy as jnp
from jax.experimental import pallas as pl
from jax.experimental.pallas import tpu as pltpu


def _vq_band_kernel(x_ref, cb_ref, cb_hi_ref, cb_lo_ref, out_ref):
    xb = x_ref[0, 0]
    cb = cb_ref[0]
    dots = jax.lax.dot_general(
        cb, xb, (((1,), (0,)), ((), ())),
        preferred_element_type=jnp.float32)
    cb_sq = jnp.sum(cb * cb, axis=1, keepdims=True)
    score = cb_sq - 2.0 * dots
    minval = jnp.min(score, axis=0)
    onehot = (score == minval[None, :]).astype(jnp.bfloat16)
    out = jax.lax.dot_general(
        cb_hi_ref[0], onehot, (((0,), (0,)), ((), ())),
        preferred_element_type=jnp.float32)
    out += jax.lax.dot_general(
        cb_lo_ref[0], onehot, (((0,), (0,)), ((), ())),
        preferred_element_type=jnp.float32)
    out_ref[0, 0] = out


def kernel(x, codebook):
    batch, n_band, n_chan, time = x.shape
    num_code = codebook.shape[1]
    cb_hi = codebook.astype(jnp.bfloat16)
    cb_lo = (codebook - cb_hi.astype(jnp.float32)).astype(jnp.bfloat16)
    grid = (n_band, batch)
    cb_spec = lambda n, b: (n, 0, 0)
    return pl.pallas_call(
        _vq_band_kernel,
        grid=grid,
        in_specs=[
            pl.BlockSpec((1, 1, n_chan, time), lambda n, b: (b, n, 0, 0)),
            pl.BlockSpec((1, num_code, n_chan), cb_spec),
            pl.BlockSpec((1, num_code, n_chan), cb_spec),
            pl.BlockSpec((1, num_code, n_chan), cb_spec),
        ],
        out_specs=pl.BlockSpec((1, 1, n_chan, time), lambda n, b: (b, n, 0, 0)),
        out_shape=jax.ShapeDtypeStruct(x.shape, x.dtype),
        compiler_params=pltpu.CompilerParams(
            dimension_semantics=("arbitrary", "arbitrary"),
        ),
    )(x, codebook, cb_hi, cb_lo)
